# baseline (speedup 1.0000x reference)
import functools

import jax
import jax.numpy as jnp
from jax import lax
from jax.experimental import pallas as pl
from jax.experimental.pallas import tpu as pltpu
from jax.experimental.pallas import tpu_sc as plsc

D = 768
H = 12
QL = 1536
KVL = 512
NOPE = 128
RD = 64
VD = 128
E = 64
TOPK = 2
FF = 256
S = 2048
B = 1
EPS = 1e-5
BASE = 10000.0
C = 64
NSLOT = E * C
TB = 256
NTB = S // TB
DH = NOPE + RD
NW = 32
TPW = S // NW
SCALE = 1.0 / float(DH) ** 0.5


def _rms(x, w):
    return x * lax.rsqrt(jnp.mean(x * x, axis=-1, keepdims=True) + EPS) * w


def _mm(a, b, dims=None):
    if dims is None:
        dims = (((a.ndim - 1,), (0,)), ((), ()))
    return lax.dot_general(a.astype(jnp.bfloat16), b.astype(jnp.bfloat16),
                           dims, preferred_element_type=jnp.float32)


def _mmw(a, b16):
    return lax.dot_general(a.astype(jnp.bfloat16), b16,
                           (((a.ndim - 1,), (0,)), ((), ())),
                           preferred_element_type=jnp.float32)


def _qkv_body(cos_ref, sin_ref, x_ref, n1_ref, wqa_ref, qaln_ref, wqb_ref,
              wkva_ref, kvln_ref, wkvb_ref, qf_ref, kf_ref, v_ref):
    x = x_ref[...]
    h = _rms(x, n1_ref[...])
    ql = _mmw(h, wqa_ref[...])
    ql = _rms(ql, qaln_ref[...])
    q = _mmw(ql, wqb_ref[...])
    cos = cos_ref[...]
    sin = sin_ref[...]

    def rope(t):
        rot = jnp.concatenate([-t[:, RD // 2:], t[:, :RD // 2]], axis=-1)
        return t * cos + rot * sin

    kv = _mmw(h, wkva_ref[...])
    kpe = rope(kv[:, KVL:]).astype(jnp.bfloat16)
    ckv = _mmw(_rms(kv[:, :KVL], kvln_ref[...]), wkvb_ref[...])
    for hh in range(H):
        qb = hh * DH
        cb = hh * (NOPE + VD)
        qf_ref[:, qb:qb + NOPE] = (q[:, qb:qb + NOPE] * SCALE).astype(jnp.bfloat16)
        qf_ref[:, qb + NOPE:qb + DH] = (rope(q[:, qb + NOPE:qb + DH]) * SCALE).astype(jnp.bfloat16)
        kf_ref[:, qb:qb + NOPE] = ckv[:, cb:cb + NOPE].astype(jnp.bfloat16)
        kf_ref[:, qb + NOPE:qb + DH] = kpe
        v_ref[:, hh * VD:(hh + 1) * VD] = ckv[:, cb + NOPE:cb + NOPE + VD].astype(jnp.bfloat16)


def _qkv(cos_t, sin_t, x2d, norm1_w, wq_a, q_a_ln_w, wq_b, wkv_a, kv_a_ln_w, wkv_b):
    full = lambda shape: pl.BlockSpec(shape, lambda i: (0,) * len(shape))
    tok = lambda w: pl.BlockSpec((TB, w), lambda i: (i, 0))
    return pl.pallas_call(
        _qkv_body,
        grid=(NTB,),
        in_specs=[tok(RD), tok(RD), tok(D), full((D,)), full((D, QL)),
                  full((QL,)), full((QL, H * DH)), full((D, KVL + RD)),
                  full((KVL,)), full((KVL, H * (NOPE + VD)))],
        out_specs=[tok(H * DH), tok(H * DH), tok(H * VD)],
        out_shape=[jax.ShapeDtypeStruct((S, H * DH), jnp.bfloat16),
                   jax.ShapeDtypeStruct((S, H * DH), jnp.bfloat16),
                   jax.ShapeDtypeStruct((S, H * VD), jnp.bfloat16)],
    )(cos_t, sin_t, x2d, norm1_w, wq_a, q_a_ln_w, wq_b, wkv_a, kv_a_ln_w, wkv_b)


TBQ = 512


def _attn_body(qf_ref, kf_ref, v_ref, o_ref):
    for h in range(H):
        q = qf_ref[:, h * DH:(h + 1) * DH]
        k = kf_ref[:, h * DH:(h + 1) * DH]
        s = lax.dot_general(q, k, (((1,), (1,)), ((), ())),
                            preferred_element_type=jnp.float32)
        m = jnp.max(s, axis=1, keepdims=True)
        p = jnp.exp((s - m).astype(jnp.bfloat16))
        l = jnp.sum(p.astype(jnp.float32), axis=1, keepdims=True)
        o = lax.dot_general(p, v_ref[:, h * VD:(h + 1) * VD],
                            (((1,), (0,)), ((), ())),
                            preferred_element_type=jnp.float32)
        o_ref[:, h * VD:(h + 1) * VD] = o / l


def _attn(qf, kf, v):
    return pl.pallas_call(
        _attn_body,
        grid=(S // TBQ,),
        in_specs=[pl.BlockSpec((TBQ, H * DH), lambda i: (i, 0)),
                  pl.BlockSpec((S, H * DH), lambda i: (0, 0)),
                  pl.BlockSpec((S, H * VD), lambda i: (0, 0))],
        out_specs=pl.BlockSpec((TBQ, H * VD), lambda i: (i, 0)),
        out_shape=jax.ShapeDtypeStruct((S, H * VD), jnp.float32),
    )(qf, kf, v)


def _router_body(x_ref, ao_ref, wo_ref, n2_ref, rw_ref,
                 x2_ref, xm_ref, i0_ref, i1_ref, w0_ref, w1_ref, hist_ref):
    x2 = x_ref[...] + lax.dot_general(ao_ref[...], wo_ref[...],
                                      (((1,), (0,)), ((), ())),
                                      preferred_element_type=jnp.float32)
    x2_ref[...] = x2
    xm = _rms(x2, n2_ref[...])
    xm_ref[...] = xm
    lg = xm @ rw_ref[...]
    iota = lax.broadcasted_iota(jnp.int32, (TB, E), 1)
    l0 = jnp.max(lg, axis=1)
    i0 = jnp.min(jnp.where(lg == l0[:, None], iota, E), axis=1)
    masked = jnp.where(iota == i0[:, None], -1e30, lg)
    l1 = jnp.max(masked, axis=1)
    i1 = jnp.min(jnp.where(masked == l1[:, None], iota, E), axis=1)
    w0 = 1.0 / (1.0 + jnp.exp(l1 - l0))
    i0_ref[0, 0, :] = i0
    i1_ref[0, 0, :] = i1
    w0_ref[0, 0, :] = w0
    w1_ref[0, 0, :] = 1.0 - w0

    @pl.when(pl.program_id(0) == 0)
    def _():
        hist_ref[...] = jnp.zeros((1, E), jnp.float32)

    oh0 = (iota == i0[:, None]).astype(jnp.float32)
    hist_ref[0, :] += jnp.sum(oh0, axis=0)


def _router(x2d, ao, wo, norm2_w, router_w):
    full = lambda shape: pl.BlockSpec(shape, lambda i: (0,) * len(shape))
    tok = lambda w: pl.BlockSpec((TB, w), lambda i: (i, 0))
    row = lambda w, dt: (pl.BlockSpec((1, 1, w), lambda i: (i, 0, 0)),
                         jax.ShapeDtypeStruct((NTB, 1, w), dt))
    specs = [row(TB, jnp.int32), row(TB, jnp.int32), row(TB, jnp.float32),
             row(TB, jnp.float32)]
    return pl.pallas_call(
        _router_body,
        grid=(NTB,),
        in_specs=[tok(D), tok(H * VD), full((H * VD, D)), full((D,)), full((D, E))],
        out_specs=[tok(D), tok(D)] + [s for s, _ in specs] +
                  [pl.BlockSpec((1, E), lambda i: (0, 0))],
        out_shape=[jax.ShapeDtypeStruct((S, D), jnp.float32),
                   jax.ShapeDtypeStruct((S, D), jnp.float32)] +
                  [t for _, t in specs] +
                  [jax.ShapeDtypeStruct((1, E), jnp.float32)],
    )(x2d, ao, wo, norm2_w, router_w)


def _pos_body(i0_ref, i1_ref, w0_ref, w1_ref, hist_ref,
              s0_ref, s1_ref, we0_ref, we1_ref, cnt0_ref, cnt1_ref):
    @pl.when(pl.program_id(0) == 0)
    def _():
        cnt0_ref[...] = jnp.zeros((1, E), jnp.float32)
        cnt1_ref[...] = hist_ref[...]

    iota = lax.broadcasted_iota(jnp.int32, (TB, E), 1)
    tri = (lax.broadcasted_iota(jnp.int32, (TB, TB), 0) >=
           lax.broadcasted_iota(jnp.int32, (TB, TB), 1)).astype(jnp.float32)

    def slot(i_ref, w_ref, cnt_ref, s_ref, we_ref):
        idx = i_ref[0, 0, :]
        oh = (iota == idx[:, None]).astype(jnp.float32)
        inc = jnp.dot(tri, oh, preferred_element_type=jnp.float32)
        pos = jnp.sum((cnt_ref[0, :][None, :] + inc - 1.0) * oh, axis=1)
        cnt_ref[0, :] += jnp.sum(oh, axis=0)
        keep = pos < float(C)
        s_ref[0, 0, :] = jnp.where(keep, idx * C + pos.astype(jnp.int32), NSLOT)
        we_ref[0, 0, :] = jnp.where(keep, w_ref[0, 0, :], 0.0)

    slot(i0_ref, w0_ref, cnt0_ref, s0_ref, we0_ref)
    slot(i1_ref, w1_ref, cnt1_ref, s1_ref, we1_ref)


def _positions(i0, i1, w0, w1, hist):
    row = lambda: pl.BlockSpec((1, 1, TB), lambda i: (i, 0, 0))
    return pl.pallas_call(
        _pos_body,
        grid=(NTB,),
        in_specs=[row(), row(), row(), row(),
                  pl.BlockSpec((1, E), lambda i: (0, 0))],
        out_specs=[row()] * 4,
        out_shape=[jax.ShapeDtypeStruct((NTB, 1, TB), jnp.int32),
                   jax.ShapeDtypeStruct((NTB, 1, TB), jnp.int32),
                   jax.ShapeDtypeStruct((NTB, 1, TB), jnp.float32),
                   jax.ShapeDtypeStruct((NTB, 1, TB), jnp.float32)],
        scratch_shapes=[pltpu.VMEM((1, E), jnp.float32),
                        pltpu.VMEM((1, E), jnp.float32)],
    )(i0, i1, w0, w1, hist)


@functools.cache
def _sc_mesh():
    return plsc.VectorSubcoreMesh(core_axis_name="c", subcore_axis_name="s")


@functools.cache
def _dispatch_kernel():
    @functools.partial(
        pl.kernel,
        out_type=jax.ShapeDtypeStruct((NSLOT + 1, D), jnp.float32),
        mesh=_sc_mesh(),
        scratch_types=[pltpu.VMEM((TPW,), jnp.int32),
                       pltpu.VMEM((TPW,), jnp.int32),
                       pltpu.VMEM((TPW, D), jnp.float32),
                       pltpu.SemaphoreType.DMA,
                       pltpu.SemaphoreType.DMA],
    )
    def body(xm_hbm, s0_hbm, s1_hbm, ein_hbm, idx0_v, idx1_v, rows_v, sem0, sem1):
        wid = lax.axis_index("s") * 2 + lax.axis_index("c")
        base = wid * TPW
        pltpu.sync_copy(s0_hbm.at[pl.ds(base, TPW)], idx0_v)
        pltpu.sync_copy(s1_hbm.at[pl.ds(base, TPW)], idx1_v)
        pltpu.sync_copy(xm_hbm.at[pl.ds(base, TPW)], rows_v)
        c0 = pltpu.async_copy(rows_v, ein_hbm.at[idx0_v], sem0)
        c1 = pltpu.async_copy(rows_v, ein_hbm.at[idx1_v], sem1)
        c0.wait()
        c1.wait()
    return body


def _dispatch(xm, s0, s1):
    return _dispatch_kernel()(xm, s0, s1)


EB = 8


def _ffn_body(ein_ref, wg_ref, wu_ref, wd_ref, eo_ref):
    for e in range(EB):
        a = ein_ref[e * C:(e + 1) * C, :]
        g = _mm(a, wg_ref[e])
        u = _mm(a, wu_ref[e])
        g = g * (1.0 / (1.0 + jnp.exp(-g)))
        eo_ref[e * C:(e + 1) * C, :] = _mm(g * u, wd_ref[e])


def _ffn(ein, w_gate, w_up, w_down):
    return pl.pallas_call(
        _ffn_body,
        grid=(E // EB,),
        in_specs=[pl.BlockSpec((EB * C, D), lambda e: (e, 0)),
                  pl.BlockSpec((EB, D, FF), lambda e: (e, 0, 0)),
                  pl.BlockSpec((EB, D, FF), lambda e: (e, 0, 0)),
                  pl.BlockSpec((EB, FF, D), lambda e: (e, 0, 0))],
        out_specs=pl.BlockSpec((EB * C, D), lambda e: (e, 0)),
        out_shape=jax.ShapeDtypeStruct((NSLOT + 1, D), jnp.float32),
    )(ein, w_gate, w_up, w_down)


@functools.cache
def _combine_kernel():
    @functools.partial(
        pl.kernel,
        out_type=(jax.ShapeDtypeStruct((S, D), jnp.float32),
                  jax.ShapeDtypeStruct((S, D), jnp.float32)),
        mesh=_sc_mesh(),
        scratch_types=[pltpu.VMEM((TPW,), jnp.int32),
                       pltpu.VMEM((TPW, D), jnp.float32),
                       pltpu.SemaphoreType.DMA],
    )
    def body(eo_hbm, s0_hbm, s1_hbm, g0_hbm, g1_hbm, idx_v, rows_v, sem):
        wid = lax.axis_index("s") * 2 + lax.axis_index("c")
        base = wid * TPW
        pltpu.sync_copy(s0_hbm.at[pl.ds(base, TPW)], idx_v)
        pltpu.async_copy(eo_hbm.at[idx_v], rows_v, sem).wait()
        pltpu.sync_copy(rows_v, g0_hbm.at[pl.ds(base, TPW)])
        pltpu.sync_copy(s1_hbm.at[pl.ds(base, TPW)], idx_v)
        pltpu.async_copy(eo_hbm.at[idx_v], rows_v, sem).wait()
        pltpu.sync_copy(rows_v, g1_hbm.at[pl.ds(base, TPW)])
    return body


def _combine(eo, s0, s1):
    return _combine_kernel()(eo, s0, s1)


def _final_body(x2_ref, g0_ref, g1_ref, we0_ref, we1_ref, o_ref):
    we0 = we0_ref[0, 0, :][:, None]
    we1 = we1_ref[0, 0, :][:, None]
    y0 = jnp.where(we0 > 0, we0 * g0_ref[...], 0.0)
    y1 = jnp.where(we1 > 0, we1 * g1_ref[...], 0.0)
    o_ref[...] = x2_ref[...] + y0 + y1


def _final(x2, g0, g1, we0, we1):
    tok = pl.BlockSpec((TB, D), lambda i: (i, 0))
    row = pl.BlockSpec((1, 1, TB), lambda i: (i, 0, 0))
    return pl.pallas_call(
        _final_body,
        grid=(NTB,),
        in_specs=[tok, tok, tok, row, row],
        out_specs=tok,
        out_shape=jax.ShapeDtypeStruct((S, D), jnp.float32),
    )(x2, g0, g1, we0, we1)


def kernel(x, norm1_w, wq_a, q_a_ln_w, wq_b, wkv_a, kv_a_ln_w, wkv_b, wo,
           norm2_w, router_w, w_gate, w_up, w_down):
    x2d = x.reshape(S, D)

    inv = 1.0 / (BASE ** (jnp.arange(0, RD, 2, dtype=jnp.float32) / RD))
    f = jnp.arange(S, dtype=jnp.float32)[:, None] * inv[None, :]
    cos_t = jnp.concatenate([jnp.cos(f), jnp.cos(f)], axis=-1)
    sin_t = jnp.concatenate([jnp.sin(f), jnp.sin(f)], axis=-1)

    bf = jnp.bfloat16
    qf, kf, v = _qkv(cos_t, sin_t, x2d, norm1_w, wq_a.astype(bf), q_a_ln_w,
                     wq_b.astype(bf), wkv_a.astype(bf), kv_a_ln_w,
                     wkv_b.astype(bf))
    ao = _attn(qf, kf, v)
    x2, xm, i0, i1, w0, w1, hist = _router(x2d, ao, wo.astype(bf), norm2_w, router_w)
    s0, s1, we0, we1 = _positions(i0, i1, w0, w1, hist)
    ein = _dispatch(xm, s0.reshape(S), s1.reshape(S))
    eo = _ffn(ein, w_gate, w_up, w_down)
    g0, g1 = _combine(eo, s0.reshape(S), s1.reshape(S))
    out = _final(x2, g0, g1, we0, we1)
    return out.reshape(S, B, D)


# speedup vs baseline: 1.0097x; 1.0097x over previous
import functools

import jax
import jax.numpy as jnp
from jax import lax
from jax.experimental import pallas as pl
from jax.experimental.pallas import tpu as pltpu
from jax.experimental.pallas import tpu_sc as plsc

D = 768
H = 12
QL = 1536
KVL = 512
NOPE = 128
RD = 64
VD = 128
E = 64
TOPK = 2
FF = 256
S = 2048
B = 1
EPS = 1e-5
BASE = 10000.0
C = 64
NSLOT = E * C
TB = 512
NTB = S // TB
DH = NOPE + RD
NW = 32
TPW = S // NW
SCALE = 1.0 / float(DH) ** 0.5


def _rms(x, w):
    return x * lax.rsqrt(jnp.mean(x * x, axis=-1, keepdims=True) + EPS) * w


def _mm(a, b, dims=None):
    if dims is None:
        dims = (((a.ndim - 1,), (0,)), ((), ()))
    return lax.dot_general(a.astype(jnp.bfloat16), b.astype(jnp.bfloat16),
                           dims, preferred_element_type=jnp.float32)


def _mmw(a, b16):
    return lax.dot_general(a.astype(jnp.bfloat16), b16,
                           (((a.ndim - 1,), (0,)), ((), ())),
                           preferred_element_type=jnp.float32)


def _qkv_body(cos_ref, sin_ref, x_ref, n1_ref, wqa_ref, qaln_ref, wqb_ref,
              wkva_ref, kvln_ref, wkvb_ref, qf_ref, kf_ref, v_ref):
    x = x_ref[...]
    h = _rms(x, n1_ref[...])
    ql = _mmw(h, wqa_ref[...])
    ql = _rms(ql, qaln_ref[...])
    q = _mmw(ql, wqb_ref[...])
    cos = cos_ref[...]
    sin = sin_ref[...]

    def rope(t):
        rot = jnp.concatenate([-t[:, RD // 2:], t[:, :RD // 2]], axis=-1)
        return t * cos + rot * sin

    kv = _mmw(h, wkva_ref[...])
    kpe = rope(kv[:, KVL:]).astype(jnp.bfloat16)
    ckv = _mmw(_rms(kv[:, :KVL], kvln_ref[...]), wkvb_ref[...])
    for hh in range(H):
        qb = hh * DH
        cb = hh * (NOPE + VD)
        qf_ref[:, qb:qb + NOPE] = (q[:, qb:qb + NOPE] * SCALE).astype(jnp.bfloat16)
        qf_ref[:, qb + NOPE:qb + DH] = (rope(q[:, qb + NOPE:qb + DH]) * SCALE).astype(jnp.bfloat16)
        kf_ref[:, qb:qb + NOPE] = ckv[:, cb:cb + NOPE].astype(jnp.bfloat16)
        kf_ref[:, qb + NOPE:qb + DH] = kpe
        v_ref[:, hh * VD:(hh + 1) * VD] = ckv[:, cb + NOPE:cb + NOPE + VD].astype(jnp.bfloat16)


def _qkv(cos_t, sin_t, x2d, norm1_w, wq_a, q_a_ln_w, wq_b, wkv_a, kv_a_ln_w, wkv_b):
    full = lambda shape: pl.BlockSpec(shape, lambda i: (0,) * len(shape))
    tok = lambda w: pl.BlockSpec((TB, w), lambda i: (i, 0))
    return pl.pallas_call(
        _qkv_body,
        grid=(NTB,),
        in_specs=[tok(RD), tok(RD), tok(D), full((D,)), full((D, QL)),
                  full((QL,)), full((QL, H * DH)), full((D, KVL + RD)),
                  full((KVL,)), full((KVL, H * (NOPE + VD)))],
        out_specs=[tok(H * DH), tok(H * DH), tok(H * VD)],
        out_shape=[jax.ShapeDtypeStruct((S, H * DH), jnp.bfloat16),
                   jax.ShapeDtypeStruct((S, H * DH), jnp.bfloat16),
                   jax.ShapeDtypeStruct((S, H * VD), jnp.bfloat16)],
    )(cos_t, sin_t, x2d, norm1_w, wq_a, q_a_ln_w, wq_b, wkv_a, kv_a_ln_w, wkv_b)


TBQ = 512


def _attn_body(qf_ref, kf_ref, v_ref, o_ref):
    for h in range(H):
        q = qf_ref[:, h * DH:(h + 1) * DH]
        k = kf_ref[:, h * DH:(h + 1) * DH]
        s = lax.dot_general(q, k, (((1,), (1,)), ((), ())),
                            preferred_element_type=jnp.float32)
        m = jnp.max(s, axis=1, keepdims=True)
        p = jnp.exp((s - m).astype(jnp.bfloat16))
        l = jnp.sum(p.astype(jnp.float32), axis=1, keepdims=True)
        o = lax.dot_general(p, v_ref[:, h * VD:(h + 1) * VD],
                            (((1,), (0,)), ((), ())),
                            preferred_element_type=jnp.float32)
        o_ref[:, h * VD:(h + 1) * VD] = o / l


def _attn(qf, kf, v):
    return pl.pallas_call(
        _attn_body,
        grid=(S // TBQ,),
        in_specs=[pl.BlockSpec((TBQ, H * DH), lambda i: (i, 0)),
                  pl.BlockSpec((S, H * DH), lambda i: (0, 0)),
                  pl.BlockSpec((S, H * VD), lambda i: (0, 0))],
        out_specs=pl.BlockSpec((TBQ, H * VD), lambda i: (i, 0)),
        out_shape=jax.ShapeDtypeStruct((S, H * VD), jnp.float32),
    )(qf, kf, v)


def _router_body(x_ref, ao_ref, wo_ref, n2_ref, rw_ref,
                 x2_ref, xm_ref, i0_ref, i1_ref, w0_ref, w1_ref, hist_ref):
    x2 = x_ref[...] + lax.dot_general(ao_ref[...], wo_ref[...],
                                      (((1,), (0,)), ((), ())),
                                      preferred_element_type=jnp.float32)
    x2_ref[...] = x2
    xm = _rms(x2, n2_ref[...])
    xm_ref[...] = xm
    lg = xm @ rw_ref[...]
    iota = lax.broadcasted_iota(jnp.int32, (TB, E), 1)
    l0 = jnp.max(lg, axis=1)
    i0 = jnp.min(jnp.where(lg == l0[:, None], iota, E), axis=1)
    masked = jnp.where(iota == i0[:, None], -1e30, lg)
    l1 = jnp.max(masked, axis=1)
    i1 = jnp.min(jnp.where(masked == l1[:, None], iota, E), axis=1)
    w0 = 1.0 / (1.0 + jnp.exp(l1 - l0))
    i0_ref[0, 0, :] = i0
    i1_ref[0, 0, :] = i1
    w0_ref[0, 0, :] = w0
    w1_ref[0, 0, :] = 1.0 - w0

    @pl.when(pl.program_id(0) == 0)
    def _():
        hist_ref[...] = jnp.zeros((1, E), jnp.float32)

    oh0 = (iota == i0[:, None]).astype(jnp.float32)
    hist_ref[0, :] += jnp.sum(oh0, axis=0)


def _router(x2d, ao, wo, norm2_w, router_w):
    full = lambda shape: pl.BlockSpec(shape, lambda i: (0,) * len(shape))
    tok = lambda w: pl.BlockSpec((TB, w), lambda i: (i, 0))
    row = lambda w, dt: (pl.BlockSpec((1, 1, w), lambda i: (i, 0, 0)),
                         jax.ShapeDtypeStruct((NTB, 1, w), dt))
    specs = [row(TB, jnp.int32), row(TB, jnp.int32), row(TB, jnp.float32),
             row(TB, jnp.float32)]
    return pl.pallas_call(
        _router_body,
        grid=(NTB,),
        in_specs=[tok(D), tok(H * VD), full((H * VD, D)), full((D,)), full((D, E))],
        out_specs=[tok(D), tok(D)] + [s for s, _ in specs] +
                  [pl.BlockSpec((1, E), lambda i: (0, 0))],
        out_shape=[jax.ShapeDtypeStruct((S, D), jnp.float32),
                   jax.ShapeDtypeStruct((S, D), jnp.float32)] +
                  [t for _, t in specs] +
                  [jax.ShapeDtypeStruct((1, E), jnp.float32)],
    )(x2d, ao, wo, norm2_w, router_w)


def _pos_body(i0_ref, i1_ref, w0_ref, w1_ref, hist_ref,
              s0_ref, s1_ref, we0_ref, we1_ref, cnt0_ref, cnt1_ref):
    @pl.when(pl.program_id(0) == 0)
    def _():
        cnt0_ref[...] = jnp.zeros((1, E), jnp.float32)
        cnt1_ref[...] = hist_ref[...]

    iota = lax.broadcasted_iota(jnp.int32, (TB, E), 1)
    tri = (lax.broadcasted_iota(jnp.int32, (TB, TB), 0) >=
           lax.broadcasted_iota(jnp.int32, (TB, TB), 1)).astype(jnp.float32)

    def slot(i_ref, w_ref, cnt_ref, s_ref, we_ref):
        idx = i_ref[0, 0, :]
        oh = (iota == idx[:, None]).astype(jnp.float32)
        inc = jnp.dot(tri, oh, preferred_element_type=jnp.float32)
        pos = jnp.sum((cnt_ref[0, :][None, :] + inc - 1.0) * oh, axis=1)
        cnt_ref[0, :] += jnp.sum(oh, axis=0)
        keep = pos < float(C)
        s_ref[0, 0, :] = jnp.where(keep, idx * C + pos.astype(jnp.int32), NSLOT)
        we_ref[0, 0, :] = jnp.where(keep, w_ref[0, 0, :], 0.0)

    slot(i0_ref, w0_ref, cnt0_ref, s0_ref, we0_ref)
    slot(i1_ref, w1_ref, cnt1_ref, s1_ref, we1_ref)


def _positions(i0, i1, w0, w1, hist):
    row = lambda: pl.BlockSpec((1, 1, TB), lambda i: (i, 0, 0))
    return pl.pallas_call(
        _pos_body,
        grid=(NTB,),
        in_specs=[row(), row(), row(), row(),
                  pl.BlockSpec((1, E), lambda i: (0, 0))],
        out_specs=[row()] * 4,
        out_shape=[jax.ShapeDtypeStruct((NTB, 1, TB), jnp.int32),
                   jax.ShapeDtypeStruct((NTB, 1, TB), jnp.int32),
                   jax.ShapeDtypeStruct((NTB, 1, TB), jnp.float32),
                   jax.ShapeDtypeStruct((NTB, 1, TB), jnp.float32)],
        scratch_shapes=[pltpu.VMEM((1, E), jnp.float32),
                        pltpu.VMEM((1, E), jnp.float32)],
    )(i0, i1, w0, w1, hist)


@functools.cache
def _sc_mesh():
    return plsc.VectorSubcoreMesh(core_axis_name="c", subcore_axis_name="s")


@functools.cache
def _dispatch_kernel():
    @functools.partial(
        pl.kernel,
        out_type=jax.ShapeDtypeStruct((NSLOT + 1, D), jnp.float32),
        mesh=_sc_mesh(),
        scratch_types=[pltpu.VMEM((TPW,), jnp.int32),
                       pltpu.VMEM((TPW,), jnp.int32),
                       pltpu.VMEM((TPW, D), jnp.float32),
                       pltpu.SemaphoreType.DMA,
                       pltpu.SemaphoreType.DMA],
    )
    def body(xm_hbm, s0_hbm, s1_hbm, ein_hbm, idx0_v, idx1_v, rows_v, sem0, sem1):
        wid = lax.axis_index("s") * 2 + lax.axis_index("c")
        base = wid * TPW
        pltpu.sync_copy(s0_hbm.at[pl.ds(base, TPW)], idx0_v)
        pltpu.sync_copy(s1_hbm.at[pl.ds(base, TPW)], idx1_v)
        pltpu.sync_copy(xm_hbm.at[pl.ds(base, TPW)], rows_v)
        c0 = pltpu.async_copy(rows_v, ein_hbm.at[idx0_v], sem0)
        c1 = pltpu.async_copy(rows_v, ein_hbm.at[idx1_v], sem1)
        c0.wait()
        c1.wait()
    return body


def _dispatch(xm, s0, s1):
    return _dispatch_kernel()(xm, s0, s1)


EB = 4


def _ffn_body(ein_ref, wg_ref, wu_ref, wd_ref, eo_ref):
    for e in range(EB):
        a = ein_ref[e * C:(e + 1) * C, :]
        g = _mm(a, wg_ref[e])
        u = _mm(a, wu_ref[e])
        g = g * (1.0 / (1.0 + jnp.exp(-g)))
        eo_ref[e * C:(e + 1) * C, :] = _mm(g * u, wd_ref[e])


def _ffn(ein, w_gate, w_up, w_down):
    return pl.pallas_call(
        _ffn_body,
        grid=(E // EB,),
        in_specs=[pl.BlockSpec((EB * C, D), lambda e: (e, 0)),
                  pl.BlockSpec((EB, D, FF), lambda e: (e, 0, 0)),
                  pl.BlockSpec((EB, D, FF), lambda e: (e, 0, 0)),
                  pl.BlockSpec((EB, FF, D), lambda e: (e, 0, 0))],
        out_specs=pl.BlockSpec((EB * C, D), lambda e: (e, 0)),
        out_shape=jax.ShapeDtypeStruct((NSLOT + 1, D), jnp.float32),
    )(ein, w_gate, w_up, w_down)


@functools.cache
def _combine_kernel():
    @functools.partial(
        pl.kernel,
        out_type=(jax.ShapeDtypeStruct((S, D), jnp.float32),
                  jax.ShapeDtypeStruct((S, D), jnp.float32)),
        mesh=_sc_mesh(),
        scratch_types=[pltpu.VMEM((TPW,), jnp.int32),
                       pltpu.VMEM((TPW,), jnp.int32),
                       pltpu.VMEM((TPW, D), jnp.float32),
                       pltpu.VMEM((TPW, D), jnp.float32),
                       pltpu.SemaphoreType.DMA,
                       pltpu.SemaphoreType.DMA],
    )
    def body(eo_hbm, s0_hbm, s1_hbm, g0_hbm, g1_hbm, idx0_v, idx1_v,
             rows0_v, rows1_v, sem0, sem1):
        wid = lax.axis_index("s") * 2 + lax.axis_index("c")
        base = wid * TPW
        pltpu.sync_copy(s0_hbm.at[pl.ds(base, TPW)], idx0_v)
        pltpu.sync_copy(s1_hbm.at[pl.ds(base, TPW)], idx1_v)
        c0 = pltpu.async_copy(eo_hbm.at[idx0_v], rows0_v, sem0)
        c1 = pltpu.async_copy(eo_hbm.at[idx1_v], rows1_v, sem1)
        c0.wait()
        pltpu.sync_copy(rows0_v, g0_hbm.at[pl.ds(base, TPW)])
        c1.wait()
        pltpu.sync_copy(rows1_v, g1_hbm.at[pl.ds(base, TPW)])
    return body


def _combine(eo, s0, s1):
    return _combine_kernel()(eo, s0, s1)


def _final_body(x2_ref, g0_ref, g1_ref, we0_ref, we1_ref, o_ref):
    we0 = we0_ref[0, 0, :][:, None]
    we1 = we1_ref[0, 0, :][:, None]
    y0 = jnp.where(we0 > 0, we0 * g0_ref[...], 0.0)
    y1 = jnp.where(we1 > 0, we1 * g1_ref[...], 0.0)
    o_ref[...] = x2_ref[...] + y0 + y1


def _final(x2, g0, g1, we0, we1):
    tok = pl.BlockSpec((TB, D), lambda i: (i, 0))
    row = pl.BlockSpec((1, 1, TB), lambda i: (i, 0, 0))
    return pl.pallas_call(
        _final_body,
        grid=(NTB,),
        in_specs=[tok, tok, tok, row, row],
        out_specs=tok,
        out_shape=jax.ShapeDtypeStruct((S, D), jnp.float32),
    )(x2, g0, g1, we0, we1)


def kernel(x, norm1_w, wq_a, q_a_ln_w, wq_b, wkv_a, kv_a_ln_w, wkv_b, wo,
           norm2_w, router_w, w_gate, w_up, w_down):
    x2d = x.reshape(S, D)

    inv = 1.0 / (BASE ** (jnp.arange(0, RD, 2, dtype=jnp.float32) / RD))
    f = jnp.arange(S, dtype=jnp.float32)[:, None] * inv[None, :]
    cos_t = jnp.concatenate([jnp.cos(f), jnp.cos(f)], axis=-1)
    sin_t = jnp.concatenate([jnp.sin(f), jnp.sin(f)], axis=-1)

    bf = jnp.bfloat16
    qf, kf, v = _qkv(cos_t, sin_t, x2d, norm1_w, wq_a.astype(bf), q_a_ln_w,
                     wq_b.astype(bf), wkv_a.astype(bf), kv_a_ln_w,
                     wkv_b.astype(bf))
    ao = _attn(qf, kf, v)
    x2, xm, i0, i1, w0, w1, hist = _router(x2d, ao, wo.astype(bf), norm2_w, router_w)
    s0, s1, we0, we1 = _positions(i0, i1, w0, w1, hist)
    ein = _dispatch(xm, s0.reshape(S), s1.reshape(S))
    eo = _ffn(ein, w_gate, w_up, w_down)
    g0, g1 = _combine(eo, s0.reshape(S), s1.reshape(S))
    out = _final(x2, g0, g1, we0, we1)
    return out.reshape(S, B, D)
